# baseline (speedup 1.0000x reference)
import functools

import jax
import jax.numpy as jnp
from jax import lax
from jax.experimental import pallas as pl
from jax.experimental.pallas import tpu as pltpu
from jax.experimental.pallas import tpu_sc as plsc

K = 512
D = 64
TB = 2048

_NC = 2
_NS = 16
_NW = _NC * _NS
_CHUNK = 128
_NBUF = 4


def _tc_body(x_ref, e_ref, idx_ref, loss_ref):
    i = pl.program_id(0)
    x = x_ref[...]
    e = e_ref[...]
    s = lax.dot_general(x, e, (((1,), (1,)), ((), ())),
                        preferred_element_type=jnp.float32)
    q2 = jnp.sum(x * x, axis=1, keepdims=True)
    e2 = jnp.sum(e * e, axis=1)[None, :]
    d2 = (q2 + e2) - 2.0 * s
    dist = jnp.sqrt(jnp.maximum(d2, 0.0))
    m = jnp.min(dist, axis=1)
    idx_ref[...] = jnp.argmin(dist, axis=1).astype(jnp.int32)
    part = jnp.sum(m * m)

    @pl.when(i == 0)
    def _():
        loss_ref[0] = 0.0

    loss_ref[0] += part


def _tc_assign(x, embed_kd):
    nt = x.shape[0]
    grid = nt // TB
    return pl.pallas_call(
        _tc_body,
        grid=(grid,),
        in_specs=[
            pl.BlockSpec((TB, D), lambda i: (i, 0)),
            pl.BlockSpec((K, D), lambda i: (0, 0)),
        ],
        out_specs=[
            pl.BlockSpec((TB,), lambda i: (i,)),
            pl.BlockSpec(memory_space=pltpu.SMEM),
        ],
        out_shape=[
            jax.ShapeDtypeStruct((nt,), jnp.int32),
            jax.ShapeDtypeStruct((1,), jnp.float32),
        ],
    )(x, embed_kd)


def _sc_gather_body(idx_hbm, table_hbm, out_hbm, idx_v, rows_v, gsem, osem):
    wid = lax.axis_index("s") * _NC + lax.axis_index("c")
    per_w = idx_hbm.shape[0] // _NW
    nch = per_w // _CHUNK
    base = pl.multiple_of(wid * per_w, _CHUNK)
    gh = [None] * nch
    wh = [None] * nch

    def _start(j):
        b = j % _NBUF
        if j - _NBUF >= 0:
            wh[j - _NBUF].wait()
        off = pl.multiple_of(base + j * _CHUNK, _CHUNK)
        pltpu.sync_copy(idx_hbm.at[pl.ds(off, _CHUNK)], idx_v.at[b])
        gh[j] = pltpu.async_copy(table_hbm.at[idx_v.at[b]], rows_v.at[b], gsem)

    _start(0)
    for j in range(nch):
        if j + 1 < nch:
            _start(j + 1)
        gh[j].wait()
        off = pl.multiple_of(base + j * _CHUNK, _CHUNK)
        wh[j] = pltpu.async_copy(rows_v.at[j % _NBUF],
                                 out_hbm.at[pl.ds(off, _CHUNK)], osem)
    for j in range(max(0, nch - _NBUF), nch):
        wh[j].wait()


def _sc_gather(idx_flat, embed):
    mesh = plsc.VectorSubcoreMesh(core_axis_name="c", subcore_axis_name="s")
    fn = pl.kernel(
        _sc_gather_body,
        out_type=jax.ShapeDtypeStruct((idx_flat.shape[0], D), jnp.float32),
        mesh=mesh,
        scratch_types=[
            pltpu.VMEM((_NBUF, _CHUNK), jnp.int32),
            pltpu.VMEM((_NBUF, _CHUNK, D), jnp.float32),
            pltpu.SemaphoreType.DMA,
            pltpu.SemaphoreType.DMA,
        ],
        compiler_params=pltpu.CompilerParams(use_tc_tiling_on_sc=False),
    )
    return fn(idx_flat, embed)


def kernel(enc, embed):
    B, C, H, W = enc.shape
    x = enc.reshape(-1, D)
    nt = x.shape[0]
    half = nt // 2
    ia, la = _tc_assign(x[:half], embed)
    qa = _sc_gather(ia, embed)
    ib, lb = _tc_assign(x[half:], embed)
    qb = _sc_gather(ib, embed)
    quant_out = jnp.concatenate([qa, qb], axis=0).reshape(B, C, H, W)
    closest = jnp.concatenate([ia, ib], axis=0).reshape(B, (C * H * W) // D)
    loss = (la[0] + lb[0]) * (2.0 / (nt * D))
    return (quant_out, loss, closest)


# speedup vs baseline: 1.3948x; 1.3948x over previous
import functools

import jax
import jax.numpy as jnp
from jax import lax
from jax.experimental import pallas as pl
from jax.experimental.pallas import tpu as pltpu
from jax.experimental.pallas import tpu_sc as plsc

K = 512
D = 64
TB = 2048

_NC = 2
_NS = 16
_NW = _NC * _NS
_CHUNK = 128
_NBUF = 4


def _tc_body(x_ref, e_ref, idx_ref, loss_ref):
    i = pl.program_id(0)
    x = x_ref[...]
    e2d = e_ref[...]
    s2 = lax.dot_general(x, e2d, (((1,), (1,)), ((), ())),
                         preferred_element_type=jnp.float32)
    q2 = jnp.sum(x * x, axis=1, keepdims=True)
    h = e2d * 0.5
    e2 = jnp.sum(h * h, axis=1)[None, :]
    d2 = (q2 + e2) - s2
    dist = jnp.sqrt(jnp.maximum(d2, 0.0))
    m = jnp.min(dist, axis=1)
    idx_ref[...] = jnp.argmin(dist, axis=1).astype(jnp.int32)
    part = jnp.sum(m * m)

    @pl.when(i == 0)
    def _():
        loss_ref[0] = 0.0

    loss_ref[0] += part


def _tc_assign(x, embed_kd):
    nt = x.shape[0]
    grid = nt // TB
    return pl.pallas_call(
        _tc_body,
        grid=(grid,),
        in_specs=[
            pl.BlockSpec((TB, D), lambda i: (i, 0)),
            pl.BlockSpec((K, D), lambda i: (0, 0)),
        ],
        out_specs=[
            pl.BlockSpec((TB,), lambda i: (i,)),
            pl.BlockSpec(memory_space=pltpu.SMEM),
        ],
        out_shape=[
            jax.ShapeDtypeStruct((nt,), jnp.int32),
            jax.ShapeDtypeStruct((1,), jnp.float32),
        ],
    )(x, embed_kd)


def _sc_gather_body(idx_hbm, table_hbm, out_hbm, idx_v, rows_v, gsem, osem):
    wid = lax.axis_index("s") * _NC + lax.axis_index("c")
    per_w = idx_hbm.shape[0] // _NW
    nch = per_w // _CHUNK
    base = pl.multiple_of(wid * per_w, _CHUNK)
    gh = [None] * nch
    wh = [None] * nch

    pltpu.sync_copy(idx_hbm.at[pl.ds(base, per_w)], idx_v)

    def _start(j):
        b = j % _NBUF
        if j - _NBUF >= 0:
            wh[j - _NBUF].wait()
        gh[j] = pltpu.async_copy(
            table_hbm.at[idx_v.at[pl.ds(j * _CHUNK, _CHUNK)]], rows_v.at[b], gsem)

    _start(0)
    for j in range(nch):
        if j + 1 < nch:
            _start(j + 1)
        gh[j].wait()
        off = pl.multiple_of(base + j * _CHUNK, _CHUNK)
        wh[j] = pltpu.async_copy(rows_v.at[j % _NBUF],
                                 out_hbm.at[pl.ds(off, _CHUNK)], osem)
    for j in range(max(0, nch - _NBUF), nch):
        wh[j].wait()


def _sc_gather(idx_flat, embed):
    mesh = plsc.VectorSubcoreMesh(core_axis_name="c", subcore_axis_name="s")
    fn = pl.kernel(
        _sc_gather_body,
        out_type=jax.ShapeDtypeStruct((idx_flat.shape[0], D), jnp.float32),
        mesh=mesh,
        scratch_types=[
            pltpu.VMEM((idx_flat.shape[0] // _NW,), jnp.int32),
            pltpu.VMEM((_NBUF, _CHUNK, D), jnp.float32),
            pltpu.SemaphoreType.DMA,
            pltpu.SemaphoreType.DMA,
        ],
        compiler_params=pltpu.CompilerParams(use_tc_tiling_on_sc=False),
    )
    return fn(idx_flat, embed)


def kernel(enc, embed):
    B, C, H, W = enc.shape
    x = enc.reshape(-1, D)
    nt = x.shape[0]
    idx_flat, la = _tc_assign(x, embed + embed)
    quant_out = _sc_gather(idx_flat, embed).reshape(B, C, H, W)
    closest = idx_flat.reshape(B, (C * H * W) // D)
    loss = la[0] * (2.0 / (nt * D))
    return (quant_out, loss, closest)
